# Initial kernel scaffold; baseline (speedup 1.0000x reference)
import jax
import jax.numpy as jnp
from jax.experimental import pallas as pl


def kernel(weight, indices):
    raise NotImplementedError("write your pallas kernel here")


# speedup vs baseline: 211.3915x; 211.3915x over previous
import functools

import jax
import jax.numpy as jnp
import numpy as np
from jax import lax
from jax.experimental import pallas as pl
from jax.experimental.pallas import tpu as pltpu
from jax.experimental.pallas import tpu_sc as plsc

HASH_A = 9824516537
HASH_B = 57857966300227
HASH_P = 117130198221199
NUM_W = 1000000
BATCH = 4096
HIST = 50
DIM = 64

NUM_CORES = 2
NUM_SUBCORES = 16
LANES = 16
NUM_WORKERS = NUM_CORES * NUM_SUBCORES
BAGS_PER_WORKER = BATCH // NUM_WORKERS
GROUPS = BAGS_PER_WORKER // LANES
JCHUNK = 25
NCHUNK = HIST // JCHUNK
CHUNK_SLOTS = JCHUNK * DIM * LANES

MASK24 = (1 << 24) - 1
P_HI = HASH_P >> 24
P_LO = HASH_P & MASK24
B_HI = HASH_B >> 24
B_LO = HASH_B & MASK24
C40 = (1 << 40) % NUM_W
C32 = (1 << 32) % NUM_W
C24 = (1 << 24) % NUM_W
INV_W = float(np.float32(1.0) / np.float32(NUM_W))

_v = np.arange(1024, dtype=object)
_t0 = np.array([(int(x) * HASH_A) % HASH_P for x in _v], dtype=np.int64)
_t1 = np.array([(int(x) * 1024 * HASH_A) % HASH_P for x in _v], dtype=np.int64)
T0_LO = np.asarray(_t0 & MASK24, dtype=np.int32)
T0_HI = np.asarray(_t0 >> 24, dtype=np.int32)
T1_LO = np.asarray(_t1 & MASK24, dtype=np.int32)
T1_HI = np.asarray(_t1 >> 24, dtype=np.int32)


def _cond_sub_p(hi, lo):
    ge = (hi > P_HI) | ((hi == P_HI) & (lo >= P_LO))
    lo2 = lo - jnp.where(ge, jnp.int32(P_LO), jnp.int32(0))
    hi2 = hi - jnp.where(ge, jnp.int32(P_HI), jnp.int32(0))
    neg = lo2 < 0
    lo3 = jnp.where(neg, lo2 + (1 << 24), lo2)
    hi3 = jnp.where(neg, hi2 - 1, hi2)
    return hi3, lo3


def _mod_w(hi, lo):
    e2 = hi >> 16
    e1 = (hi >> 8) & 255
    e0 = hi & 255
    u = e2 * C40 + e1 * C32 + e0 * C24 + lo
    q = (u.astype(jnp.float32) * INV_W).astype(jnp.int32)
    r = u - q * NUM_W
    r = jnp.where(r < 0, r + NUM_W, r)
    r = jnp.where(r >= NUM_W, r - NUM_W, r)
    return r


def _sc_body(weight, idx32, t0lo, t0hi, t1lo, t1hi, out,
             bags_idx, ibuf, vbuf, outbuf, v0lo, v0hi, v1lo, v1hi, sem):
    wid = lax.axis_index("s") * NUM_CORES + lax.axis_index("c")
    lane = lax.iota(jnp.int32, LANES)

    pltpu.sync_copy(t0lo, v0lo)
    pltpu.sync_copy(t0hi, v0hi)
    pltpu.sync_copy(t1lo, v1lo)
    pltpu.sync_copy(t1hi, v1hi)

    def compute_chunk(j0):
        def j_body(j, _):
            jfull = jnp.full((LANES,), j, dtype=jnp.int32)
            i_vec = plsc.load_gather(bags_idx, [lane, jfull])
            i0 = i_vec & 1023
            i1 = i_vec >> 10
            mlo = plsc.load_gather(v0lo, [i0]) + plsc.load_gather(v1lo, [i1])
            mhi = plsc.load_gather(v0hi, [i0]) + plsc.load_gather(v1hi, [i1])
            mhi = mhi + (mlo >> 24)
            mlo = mlo & MASK24
            shi, slo = _cond_sub_p(mhi, mlo)

            def d_body(dg, carry):
                shi, slo = carry
                base = ((j - j0) * DIM + dg * 8) * LANES
                for k in range(8):
                    slot = _mod_w(shi, slo)
                    ibuf[pl.ds(base + k * LANES, LANES)] = slot
                    t = slo + B_LO
                    nhi = shi + B_HI + (t >> 24)
                    nlo = t & MASK24
                    shi, slo = _cond_sub_p(nhi, nlo)
                return shi, slo

            lax.fori_loop(jnp.int32(0), jnp.int32(DIM // 8), d_body, (shi, slo))
            return jnp.int32(0)

        lax.fori_loop(jnp.int32(j0), jnp.int32(j0 + JCHUNK), j_body, jnp.int32(0))

    def accumulate_chunk(first):
        def d_body(d, _):
            acc = vbuf[pl.ds(d * LANES, LANES)]
            for j in range(1, JCHUNK):
                acc = acc + vbuf[pl.ds((j * DIM + d) * LANES, LANES)]
            dfull = jnp.full((LANES,), d, dtype=jnp.int32)
            if first:
                plsc.store_scatter(outbuf, [lane, dfull], acc)
            else:
                plsc.addupdate_scatter(outbuf, [lane, dfull], acc)
            return jnp.int32(0)

        lax.fori_loop(jnp.int32(0), jnp.int32(DIM), d_body, jnp.int32(0))

    def group_body(g, _):
        base = wid * BAGS_PER_WORKER + g * LANES
        pltpu.sync_copy(idx32.at[pl.ds(base, LANES), :], bags_idx)
        for c in range(NCHUNK):
            compute_chunk(c * JCHUNK)
            pltpu.async_copy(weight.at[ibuf], vbuf, sem).wait()
            accumulate_chunk(first=(c == 0))
        pltpu.sync_copy(outbuf, out.at[pl.ds(base, LANES), :])
        return jnp.int32(0)

    lax.fori_loop(jnp.int32(0), jnp.int32(GROUPS), group_body, jnp.int32(0))


@jax.jit
def _hashed_embedding_bag(weight, idx32):
    mesh = plsc.VectorSubcoreMesh(core_axis_name="c", subcore_axis_name="s")
    f = pl.kernel(
        _sc_body,
        out_type=jax.ShapeDtypeStruct((BATCH, DIM), jnp.float32),
        mesh=mesh,
        compiler_params=pltpu.CompilerParams(needs_layout_passes=False),
        scratch_types=[
            pltpu.VMEM((LANES, HIST), jnp.int32),
            pltpu.VMEM((CHUNK_SLOTS,), jnp.int32),
            pltpu.VMEM((CHUNK_SLOTS,), jnp.float32),
            pltpu.VMEM((LANES, DIM), jnp.float32),
            pltpu.VMEM((1024,), jnp.int32),
            pltpu.VMEM((1024,), jnp.int32),
            pltpu.VMEM((1024,), jnp.int32),
            pltpu.VMEM((1024,), jnp.int32),
            pltpu.SemaphoreType.DMA,
        ],
    )
    return f(weight, idx32,
             jnp.asarray(T0_LO), jnp.asarray(T0_HI),
             jnp.asarray(T1_LO), jnp.asarray(T1_HI))


def kernel(weight, indices):
    weight = weight.astype(jnp.float32)
    idx32 = indices.astype(jnp.int32)
    return _hashed_embedding_bag(weight, idx32)


# speedup vs baseline: 291.5330x; 1.3791x over previous
import functools

import jax
import jax.numpy as jnp
import numpy as np
from jax import lax
from jax.experimental import pallas as pl
from jax.experimental.pallas import tpu as pltpu
from jax.experimental.pallas import tpu_sc as plsc

HASH_A = 9824516537
HASH_B = 57857966300227
HASH_P = 117130198221199
NUM_W = 1000000
BATCH = 4096
HIST = 50
DIM = 64

NUM_CORES = 2
NUM_SUBCORES = 16
LANES = 16
NUM_WORKERS = NUM_CORES * NUM_SUBCORES
BAGS_PER_WORKER = BATCH // NUM_WORKERS
GROUPS = BAGS_PER_WORKER // LANES
JCHUNK = 25
NCHUNK = HIST // JCHUNK
CHUNK_SLOTS = JCHUNK * DIM * LANES

MASK24 = (1 << 24) - 1
P_HI = HASH_P >> 24
P_LO = HASH_P & MASK24
B_HI = HASH_B >> 24
B_LO = HASH_B & MASK24
C40 = (1 << 40) % NUM_W
C32 = (1 << 32) % NUM_W
C24 = (1 << 24) % NUM_W
INV_W = float(np.float32(1.0) / np.float32(NUM_W))

_v = np.arange(1024, dtype=object)
_t0 = np.array([(int(x) * HASH_A) % HASH_P for x in _v], dtype=np.int64)
_t1 = np.array([(int(x) * 1024 * HASH_A) % HASH_P for x in _v], dtype=np.int64)
T0_LO = np.asarray(_t0 & MASK24, dtype=np.int32)
T0_HI = np.asarray(_t0 >> 24, dtype=np.int32)
T1_LO = np.asarray(_t1 & MASK24, dtype=np.int32)
T1_HI = np.asarray(_t1 >> 24, dtype=np.int32)


def _cond_sub_p(hi, lo):
    ge = (hi > P_HI) | ((hi == P_HI) & (lo >= P_LO))
    lo2 = lo - jnp.where(ge, jnp.int32(P_LO), jnp.int32(0))
    hi2 = hi - jnp.where(ge, jnp.int32(P_HI), jnp.int32(0))
    neg = lo2 < 0
    lo3 = jnp.where(neg, lo2 + (1 << 24), lo2)
    hi3 = jnp.where(neg, hi2 - 1, hi2)
    return hi3, lo3


def _mod_w(hi, lo):
    e2 = hi >> 16
    e1 = (hi >> 8) & 255
    e0 = hi & 255
    u = e2 * C40 + e1 * C32 + e0 * C24 + lo
    q = (u.astype(jnp.float32) * INV_W).astype(jnp.int32)
    r = u - q * NUM_W
    r = jnp.where(r < 0, r + NUM_W, r)
    r = jnp.where(r >= NUM_W, r - NUM_W, r)
    return r


def _sc_body(weight, idx32, t0lo, t0hi, t1lo, t1hi, out,
             bags_idx, ibuf, vbuf, outbuf, v0lo, v0hi, v1lo, v1hi, sem):
    wid = lax.axis_index("s") * NUM_CORES + lax.axis_index("c")
    lane = lax.iota(jnp.int32, LANES)

    pltpu.sync_copy(t0lo, v0lo)
    pltpu.sync_copy(t0hi, v0hi)
    pltpu.sync_copy(t1lo, v1lo)
    pltpu.sync_copy(t1hi, v1hi)

    def compute_chunk(j0, poff):
        def j_body(j, _):
            jfull = jnp.full((LANES,), j, dtype=jnp.int32)
            i_vec = plsc.load_gather(bags_idx, [lane, jfull])
            i0 = i_vec & 1023
            i1 = i_vec >> 10
            mlo = plsc.load_gather(v0lo, [i0]) + plsc.load_gather(v1lo, [i1])
            mhi = plsc.load_gather(v0hi, [i0]) + plsc.load_gather(v1hi, [i1])
            mhi = mhi + (mlo >> 24)
            mlo = mlo & MASK24
            shi, slo = _cond_sub_p(mhi, mlo)

            def d_body(dg, carry):
                shi, slo = carry
                base = poff + ((j - j0) * DIM + dg * 8) * LANES
                for k in range(8):
                    slot = _mod_w(shi, slo)
                    ibuf[pl.ds(base + k * LANES, LANES)] = slot
                    t = slo + B_LO
                    nhi = shi + B_HI + (t >> 24)
                    nlo = t & MASK24
                    shi, slo = _cond_sub_p(nhi, nlo)
                return shi, slo

            lax.fori_loop(jnp.int32(0), jnp.int32(DIM // 8), d_body, (shi, slo))
            return jnp.int32(0)

        lax.fori_loop(j0, j0 + jnp.int32(JCHUNK), j_body, jnp.int32(0))

    def accumulate_chunk(first, poff):
        def d_body(d, _):
            acc = vbuf[pl.ds(poff + d * LANES, LANES)]
            for j in range(1, JCHUNK):
                acc = acc + vbuf[pl.ds(poff + (j * DIM + d) * LANES, LANES)]
            dfull = jnp.full((LANES,), d, dtype=jnp.int32)
            if first:
                plsc.store_scatter(outbuf, [lane, dfull], acc)
            else:
                plsc.addupdate_scatter(outbuf, [lane, dfull], acc)
            return jnp.int32(0)

        lax.fori_loop(jnp.int32(0), jnp.int32(DIM), d_body, jnp.int32(0))

    def drain_prev(t):
        tp = t - 1
        pprev = tp & 1
        poff = pprev * CHUNK_SLOTS
        pltpu.make_async_copy(
            weight.at[ibuf.at[pl.ds(poff, CHUNK_SLOTS)]],
            vbuf.at[pl.ds(poff, CHUNK_SLOTS)], sem).wait()

        @pl.when((tp & 1) == 0)
        def _():
            accumulate_chunk(True, poff)

        @pl.when((tp & 1) == 1)
        def _():
            accumulate_chunk(False, poff)
            gprev = tp >> 1
            base = wid * BAGS_PER_WORKER + gprev * LANES
            pltpu.sync_copy(outbuf, out.at[pl.ds(base, LANES), :])

    def stage_body(t, _):
        p = t & 1
        poff = p * CHUNK_SLOTS

        @pl.when((t & 1) == 0)
        def _():
            base = wid * BAGS_PER_WORKER + (t >> 1) * LANES
            pltpu.sync_copy(idx32.at[pl.ds(base, LANES), :], bags_idx)

        compute_chunk((t & 1) * JCHUNK, poff)

        @pl.when(t > 0)
        def _():
            drain_prev(t)

        pltpu.async_copy(
            weight.at[ibuf.at[pl.ds(poff, CHUNK_SLOTS)]],
            vbuf.at[pl.ds(poff, CHUNK_SLOTS)], sem)
        return jnp.int32(0)

    nstages = jnp.int32(GROUPS * NCHUNK)
    lax.fori_loop(jnp.int32(0), nstages, stage_body, jnp.int32(0))
    drain_prev(nstages)


@jax.jit
def _hashed_embedding_bag(weight, idx32):
    mesh = plsc.VectorSubcoreMesh(core_axis_name="c", subcore_axis_name="s")
    f = pl.kernel(
        _sc_body,
        out_type=jax.ShapeDtypeStruct((BATCH, DIM), jnp.float32),
        mesh=mesh,
        compiler_params=pltpu.CompilerParams(needs_layout_passes=False),
        scratch_types=[
            pltpu.VMEM((LANES, HIST), jnp.int32),
            pltpu.VMEM((2 * CHUNK_SLOTS,), jnp.int32),
            pltpu.VMEM((2 * CHUNK_SLOTS,), jnp.float32),
            pltpu.VMEM((LANES, DIM), jnp.float32),
            pltpu.VMEM((1024,), jnp.int32),
            pltpu.VMEM((1024,), jnp.int32),
            pltpu.VMEM((1024,), jnp.int32),
            pltpu.VMEM((1024,), jnp.int32),
            pltpu.SemaphoreType.DMA,
        ],
    )
    return f(weight, idx32,
             jnp.asarray(T0_LO), jnp.asarray(T0_HI),
             jnp.asarray(T1_LO), jnp.asarray(T1_HI))


def kernel(weight, indices):
    weight = weight.astype(jnp.float32)
    idx32 = indices.astype(jnp.int32)
    return _hashed_embedding_bag(weight, idx32)
